# baseline (speedup 1.0000x reference)
import jax
import jax.numpy as jnp
from jax.experimental import pallas as pl
from jax.experimental.pallas import tpu as pltpu

_BIG = 400
_SMALL = 80


def _schedule(rows):
    sched = []
    off = 0
    while rows - off > _BIG:
        sched.append((off, _BIG))
        off += _BIG
    while off < rows:
        sched.append((off, _SMALL))
        off += _SMALL
    return sched


def _gcn_kernel(seq_hbm, wt_ref, bias_ref, adj_hbm, out_hbm,
                buf, stage, seq_vmem, fts, in_sem, out_sem, seq_sem):
    rows = adj_hbm.shape[0]
    sched = _schedule(rows)

    def in_copy(idx):
        off, sz = sched[idx]
        slot = idx % 2
        return pltpu.make_async_copy(
            adj_hbm.at[pl.ds(off, sz), :],
            buf.at[slot, pl.ds(0, sz), :],
            in_sem.at[slot],
        )

    def out_copy(idx):
        off, sz = sched[idx]
        slot = idx % 2
        return pltpu.make_async_copy(
            stage.at[slot, pl.ds(0, sz), :],
            out_hbm.at[pl.ds(off, sz), :],
            out_sem.at[slot],
        )

    seq_copy = pltpu.make_async_copy(seq_hbm.at[...], seq_vmem.at[...], seq_sem)
    seq_copy.start()
    in_copy(0).start()
    in_copy(1).start()

    seq_copy.wait()
    fts[...] = jnp.dot(
        seq_vmem[...],
        wt_ref[...],
        precision=jax.lax.Precision.DEFAULT,
        preferred_element_type=jnp.float32,
    )

    for idx, (off, sz) in enumerate(sched):
        slot = idx % 2
        in_copy(idx).wait()
        if idx >= 2:
            out_copy(idx - 2).wait()
        acc = jnp.dot(
            buf[slot, pl.ds(0, sz), :],
            fts[...],
            precision=jax.lax.Precision.DEFAULT,
            preferred_element_type=jnp.float32,
        )
        if idx + 2 < len(sched):
            in_copy(idx + 2).start()
        stage[slot, pl.ds(0, sz), :] = jnp.maximum(acc + bias_ref[...], 0.0)
        out_copy(idx).start()

    out_copy(len(sched) - 2).wait()
    out_copy(len(sched) - 1).wait()


def kernel(seq, adj, W, bias):
    b, n, in_ft = seq.shape
    out_ft = W.shape[0]
    rows = b * n
    seq2d = seq.reshape(rows, in_ft)
    adj2d = adj.reshape(rows, n)
    wt = W.T
    bias2d = bias.reshape(1, out_ft)

    out = pl.pallas_call(
        _gcn_kernel,
        in_specs=[
            pl.BlockSpec(memory_space=pltpu.MemorySpace.HBM),
            pl.BlockSpec(memory_space=pltpu.MemorySpace.VMEM),
            pl.BlockSpec(memory_space=pltpu.MemorySpace.VMEM),
            pl.BlockSpec(memory_space=pltpu.MemorySpace.HBM),
        ],
        out_specs=pl.BlockSpec(memory_space=pltpu.MemorySpace.HBM),
        out_shape=jax.ShapeDtypeStruct((rows, out_ft), jnp.float32),
        scratch_shapes=[
            pltpu.VMEM((2, _BIG, n), jnp.float32),
            pltpu.VMEM((2, _BIG, out_ft), jnp.float32),
            pltpu.VMEM((rows, in_ft), jnp.float32),
            pltpu.VMEM((rows, out_ft), jnp.float32),
            pltpu.SemaphoreType.DMA((2,)),
            pltpu.SemaphoreType.DMA((2,)),
            pltpu.SemaphoreType.DMA,
        ],
    )(seq2d, wt, bias2d, adj2d)

    return out.reshape(b, n, out_ft)
